# Initial kernel scaffold; baseline (speedup 1.0000x reference)
import jax
import jax.numpy as jnp
from jax.experimental import pallas as pl


def kernel(vertices, params):
    raise NotImplementedError("write your pallas kernel here")


# speedup vs baseline: 5.3717x; 5.3717x over previous
import functools

import jax
import jax.numpy as jnp
from jax import lax
from jax.experimental import pallas as pl
from jax.experimental.pallas import tpu as pltpu
from jax.experimental.pallas import tpu_sc as plsc

_SUP = 1
_K = 51
_NW = 32


def _knn_body(prow_ref, pallT_ref, idx_ref, dpos_ref, dist_ref, *, K):
    prow = prow_ref[0]
    pallT = pallT_ref[0]
    Vt = prow.shape[0]
    V = pallT.shape[1]
    inner = lax.dot_general(prow.astype(jnp.bfloat16), pallT.astype(jnp.bfloat16),
                            (((1,), (0,)), ((), ())),
                            preferred_element_type=jnp.float32)
    sqr = jnp.sum(prow * prow, axis=1)[:, None]
    sqa = jnp.sum(pallT * pallT, axis=0)[None, :]
    dist_ref[:, :] = sqr - 2.0 * inner + sqa
    iot = lax.broadcasted_iota(jnp.int32, (Vt, V), 1)

    def body(t, carry):
        d = dist_ref[:, :]
        m = jnp.min(d, axis=1, keepdims=True)
        a = jnp.min(jnp.where(d == m, iot, V), axis=1)
        idx_ref[0, pl.ds(t, 1), :] = (a + pl.program_id(0) * V)[None, :]
        onehot = iot == a[:, None]
        dist_ref[:, :] = jnp.where(onehot, jnp.float32(jnp.inf), d)
        nb = jnp.stack(
            [jnp.sum(jnp.where(onehot, pallT[c][None, :], 0.0), axis=1)
             for c in range(3)], axis=1)
        dpos_ref[0, pl.ds(t, 1), :, :] = (nb - prow)[None]
        return carry

    lax.fori_loop(0, K, body, 0)


def _knn51(pos, K=_K):
    B, V, _ = pos.shape
    Vt = min(V, 256)
    grid = (B, V // Vt)
    posT = jnp.transpose(pos, (0, 2, 1))
    idx, dpos = pl.pallas_call(
        functools.partial(_knn_body, K=K),
        grid=grid,
        in_specs=[pl.BlockSpec((1, Vt, 3), lambda b, j: (b, j, 0)),
                  pl.BlockSpec((1, 3, V), lambda b, j: (b, 0, 0))],
        out_specs=[pl.BlockSpec((1, K, Vt), lambda b, j: (b, 0, j)),
                   pl.BlockSpec((1, K, Vt, 3), lambda b, j: (b, 0, j, 0))],
        out_shape=[jax.ShapeDtypeStruct((B, K, V), jnp.int32),
                   jax.ShapeDtypeStruct((B, K, V, 3), jnp.float32)],
        scratch_shapes=[pltpu.VMEM((Vt, V), jnp.float32)],
    )(pos, posT)
    return idx, dpos


@functools.partial(jax.jit, static_argnames=("chunk",))
def _sc_gather_call(table, idx, chunk):
    M = idx.shape[0]
    D = table.shape[1]
    b_per_w = M // _NW
    n_chunks = b_per_w // chunk
    mesh = plsc.VectorSubcoreMesh(core_axis_name="c", subcore_axis_name="s")

    @functools.partial(
        pl.kernel, mesh=mesh,
        out_type=jax.ShapeDtypeStruct((M, D), jnp.float32),
        scratch_types=[
            pltpu.VMEM((b_per_w,), jnp.int32),
            pltpu.VMEM((chunk, D), jnp.float32),
            pltpu.SemaphoreType.DMA,
        ],
    )
    def k(table_hbm, idx_hbm, out_hbm, idx_v, rows_v, sem):
        wid = lax.axis_index("s") * 2 + lax.axis_index("c")
        base = wid * b_per_w
        pltpu.sync_copy(idx_hbm.at[pl.ds(base, b_per_w)], idx_v)

        def body(c, carry):
            off = c * chunk
            pltpu.async_copy(table_hbm.at[idx_v.at[pl.ds(off, chunk)]],
                             rows_v, sem).wait()
            pltpu.sync_copy(rows_v, out_hbm.at[pl.ds(base + off, chunk)])
            return carry

        lax.fori_loop(0, n_chunks, body, 0)

    return k(table, idx)


def _sc_gather(table, idx):
    M = idx.shape[0]
    D = table.shape[1]
    chunk = 64 if D > 256 else 128
    step = _NW * chunk
    M_pad = -(-M // step) * step
    if M_pad != M:
        idx = jnp.concatenate([idx, jnp.zeros((M_pad - M,), jnp.int32)])
    out = _sc_gather_call(table, idx, chunk)
    return out[:M] if M_pad != M else out


def _gather_nb(t, idx):
    B, V2, N = idx.shape
    C = t.shape[-1]
    flat = _sc_gather(t.reshape(-1, C), idx.reshape(-1))
    return flat.reshape(B, V2, N, C)


def _knn_bundle(v):
    idxT, dposT = _knn51(v)
    idx = jnp.transpose(idxT, (0, 2, 1))
    d = jnp.transpose(dposT, (0, 2, 1, 3))[:, :, 1:51]
    n = jnp.linalg.norm(d, axis=-1, keepdims=True)
    ndn50 = d / jnp.maximum(n, 1e-12)
    rel = -d[:, :, :16]
    return idx, ndn50, rel


def _conv_surface(p, ndn, kernel_num):
    dnorm = jnp.maximum(jnp.linalg.norm(p['dir'], axis=0, keepdims=True), 1e-12)
    sdn = p['dir'] / dnorm
    theta = jax.nn.relu(ndn @ sdn)
    return jnp.max(theta, axis=2)


def _conv_layer(p, idx, ndn, fmap, out_ch):
    dnorm = jnp.maximum(jnp.linalg.norm(p['dir'], axis=0, keepdims=True), 1e-12)
    sdn = p['dir'] / dnorm
    theta = jax.nn.relu(ndn @ sdn)
    fout = fmap @ p['w'] + p['b']
    fc = fout[:, :, :out_ch]
    fs = fout[:, :, out_ch:]
    fs_nb = _gather_nb(fs, idx)
    act = jnp.max(theta * fs_nb, axis=2)
    return fc + act


def _bn(p, x):
    mu = jnp.mean(x, axis=(0, 1), keepdims=True)
    var = jnp.var(x, axis=(0, 1), keepdims=True)
    return (x - mu) / jnp.sqrt(var + 1e-5) * p['g'] + p['b']


def _fusion_surface(p, knn, dim):
    idx, ndn50, _ = knn
    fl = jax.nn.relu(_bn(p['bn_l'], _conv_surface(p['conv_l'], ndn50[:, :, :10], dim)))
    fg = jax.nn.relu(_bn(p['bn_g0'], _conv_surface(p['conv_g0'], ndn50, dim)))
    fg = jax.nn.relu(_bn(p['bn_g1'], _conv_layer(p['conv_g1'], idx[:, :, 1:51],
                                                 ndn50, fg, dim)))
    return jnp.concatenate([fl, fg], axis=2)


def _fusion(p, knn, inp, dim):
    idx, ndn50, _ = knn
    fl = jax.nn.relu(_bn(p['bn_l'], _conv_layer(p['conv_l'], idx[:, :, 1:11],
                                                ndn50[:, :, :10], inp, dim)))
    fg = jax.nn.relu(_bn(p['bn_g0'], _conv_layer(p['conv_g0'], idx[:, :, 1:51],
                                                 ndn50, inp, dim)))
    fg = jax.nn.relu(_bn(p['bn_g1'], _conv_layer(p['conv_g1'], idx[:, :, 1:51],
                                                 ndn50, fg, dim)))
    return jnp.concatenate([fl, fg], axis=2)


def _linear_relu(p, x):
    return jax.nn.relu(x @ p['w'] + p['b'])


def _pool(knn, vertices, fmap, rate=4):
    idx = knn[0]
    nb = _gather_nb(fmap, idx[:, :, 1:5])
    pooled = jnp.max(nb, axis=2)
    pool_num = vertices.shape[1] // rate
    return vertices[:, :pool_num, :], pooled[:, :pool_num, :]


def _transformer(p, knn, feat):
    idx, _, rel = knn
    idx16 = idx[:, :, 1:17]
    identity = feat
    x = feat @ p['start']['w'] + p['start']['b']
    q = x @ p['q']['w'] + p['q']['b']
    k = x @ p['k']['w'] + p['k']['b']
    v = x @ p['v']['w'] + p['v']['b']
    knb = _gather_nb(k, idx16)
    vnb = _gather_nb(v, idx16)
    pe = jax.nn.relu(rel @ p['pos1']['w'] + p['pos1']['b']) @ p['pos2']['w'] + p['pos2']['b']
    a = jax.nn.relu((q[:, :, None, :] - knb + pe) @ p['attn1']['w'] + p['attn1']['b']) @ p['attn2']['w'] + p['attn2']['b']
    a = jax.nn.softmax(a, axis=2)
    agg = jnp.sum(a * (vnb + pe), axis=2)
    return agg @ p['end']['w'] + p['end']['b'] + identity


def kernel(vertices, params):
    v = jnp.transpose(vertices, (0, 2, 1))
    knn0 = _knn_bundle(v)

    fm0 = _fusion_surface(params['conv_0'], knn0, 128)
    fm0 = _linear_relu(params['down0'], fm0)
    fm0 = _transformer(params['att0'], knn0, fm0)
    fm1 = _fusion(params['conv_1'], knn0, fm0, 128)
    fm1 = _linear_relu(params['down1'], fm1)
    fm1 = _transformer(params['att1'], knn0, fm1)
    vp1, fp1 = _pool(knn0, v, fm1)

    knn1 = _knn_bundle(vp1)

    fm2 = _fusion(params['conv_2'], knn1, fp1, 128)
    fm2 = _transformer(params['att2'], knn1, fm2)
    fm3 = _fusion(params['conv_3'], knn1, fm2, 256)
    fm3 = _transformer(params['att3'], knn1, fm3)
    vp2, fp2 = _pool(knn1, vp1, fm3)

    knn2 = _knn_bundle(vp2)

    fm4 = _fusion(params['conv_4'], knn2, fp2, 512)
    fm4 = _linear_relu(params['down2'], fm4)
    fm4 = _transformer(params['att4'], knn2, fm4)
    return jnp.max(fm4, axis=1)
